# baseline (speedup 1.0000x reference)
import functools

import jax
import jax.numpy as jnp
from jax import lax
from jax.experimental import pallas as pl
from jax.experimental.pallas import tpu as pltpu
from jax.experimental.pallas import tpu_sc as plsc

B = 4096
L = 200
D_IN = 64
STATION_DIM = 16
SEASON_DIM = 4
N_SEASONS = 4
D_OUT = D_IN + STATION_DIM + SEASON_DIM

_NC = 2
_NS = 16
_NW = _NC * _NS
_B_PER_W = B // _NW


def _sc_assemble(x, station_ids, season_ids, station_table, season_table):
    mesh = plsc.VectorSubcoreMesh(core_axis_name="c", subcore_axis_name="s")

    @functools.partial(
        pl.kernel,
        mesh=mesh,
        out_type=jax.ShapeDtypeStruct((B, L, D_OUT), jnp.float32),
        scratch_types=[
            pltpu.VMEM((_B_PER_W,), jnp.int32),
            pltpu.VMEM((_B_PER_W, STATION_DIM), jnp.float32),
            pltpu.VMEM((L, D_IN), jnp.float32),
            pltpu.VMEM((L, STATION_DIM), jnp.float32),
            pltpu.VMEM((L, SEASON_DIM), jnp.float32),
            pltpu.VMEM((N_SEASONS, SEASON_DIM), jnp.float32),
            pltpu.VMEM((_B_PER_W,), jnp.int32),
            pltpu.SemaphoreType.DMA,
        ],
        compiler_params=pltpu.CompilerParams(use_tc_tiling_on_sc=False,
                                             needs_layout_passes=False),
    )
    def k(x_hbm, sid_hbm, seid_hbm, table_hbm, stab_hbm, out_hbm,
          idx_v, st_rows, xbuf, st_bc, se_bc, stab_v, sed_v, sem):
        wid = lax.axis_index("s") * _NC + lax.axis_index("c")
        base = wid * _B_PER_W
        pltpu.sync_copy(sid_hbm.at[pl.ds(base, _B_PER_W)], idx_v)
        pltpu.async_copy(table_hbm.at[idx_v], st_rows, sem).wait()
        pltpu.sync_copy(seid_hbm.at[pl.ds(base, _B_PER_W)], sed_v)
        pltpu.sync_copy(stab_hbm, stab_v)

        lanes = lax.iota(jnp.int32, 16)

        def body(j, carry):
            b = base + j
            pltpu.sync_copy(x_hbm.at[b], xbuf)
            pltpu.sync_copy(xbuf, out_hbm.at[b, :, 0:D_IN])
            stv = st_rows[j, :]
            def fill_l(l, c0):
                st_bc[l, :] = stv
                return c0
            lax.fori_loop(0, L, fill_l, 0)
            pltpu.sync_copy(st_bc, out_hbm.at[b, :, D_IN:D_IN + STATION_DIM])
            sid_splat = plsc.load_gather(
                sed_v, [jnp.full((16,), j, jnp.int32)])
            p = plsc.load_gather(stab_v, [sid_splat, lanes & 3])
            def fill_t(t, c0):
                flat = t * 16 + lanes
                plsc.store_scatter(se_bc, [flat >> 2, flat & 3], p)
                return c0
            lax.fori_loop(0, L * SEASON_DIM // 16, fill_t, 0)
            pltpu.sync_copy(se_bc,
                            out_hbm.at[b, :, D_IN + STATION_DIM:D_OUT])
            return carry
        lax.fori_loop(0, _B_PER_W, body, 0)

    return k(x, station_ids, season_ids, station_table, season_table)


def kernel(x, station_ids, season_ids, station_table, season_table):
    return _sc_assemble(x, station_ids, season_ids, station_table,
                        season_table)
